# baseline (speedup 1.0000x reference)
import jax
import jax.numpy as jnp
from jax import lax
from jax.experimental import pallas as pl
from jax.experimental.pallas import tpu as pltpu
from jax.experimental.pallas import tpu_sc as plsc

_VOCAB_SIZE = 1000
_TBL = 2048
_VOCAB_PAD = 1024
_N = 4096 * 20
_NW = 32
_PER_W = _N // _NW
_L = 16


def _lookup_body(inp_hbm, vocab_hbm, out_hbm, inp_v, vocab_v, inv_v, out_v):
    wid = lax.axis_index("s") * 2 + lax.axis_index("c")
    base = wid * (2 * _PER_W)
    pltpu.sync_copy(inp_hbm.at[pl.ds(base, 2 * _PER_W)], inp_v)
    pltpu.sync_copy(vocab_hbm, vocab_v)

    lane = lax.iota(jnp.int32, _L)
    lane2 = lane * 2
    zeros = jnp.zeros((_L,), jnp.int32)
    oov = jnp.full((_L,), _VOCAB_SIZE, jnp.int32)

    def init_step(i, carry):
        inv_v[pl.ds(i * _L, _L)] = oov
        return carry

    lax.fori_loop(0, _TBL // _L, init_step, 0, unroll=8)

    def scatter_step(j, carry):
        keys = plsc.load_gather(vocab_v, [lane2 + j * (2 * _L)])
        plsc.store_scatter(inv_v, [keys], lane + j * _L)
        return carry

    lax.fori_loop(0, _VOCAB_PAD // _L, scatter_step, 0, unroll=8)

    def gather_step(i, carry):
        off = i * (2 * _L)
        toks = plsc.load_gather(inp_v, [lane2 + off])
        vals = plsc.load_gather(inv_v, [toks])
        plsc.store_scatter(out_v, [lane2 + off], vals)
        plsc.store_scatter(out_v, [lane2 + off + 1], zeros)
        return carry

    lax.fori_loop(0, _PER_W // _L, gather_step, 0, unroll=8)

    pltpu.sync_copy(out_v, out_hbm.at[pl.ds(base, 2 * _PER_W)])


@jax.jit
def _lookup(flat_pairs, vocab_pairs):
    with jax.enable_x64(False):
        mesh = plsc.VectorSubcoreMesh(core_axis_name="c", subcore_axis_name="s")
        run = pl.kernel(
            _lookup_body,
            out_type=jax.ShapeDtypeStruct((2 * _N,), jnp.int32),
            mesh=mesh,
            scratch_types=[
                pltpu.VMEM((2 * _PER_W,), jnp.int32),
                pltpu.VMEM((2 * _VOCAB_PAD,), jnp.int32),
                pltpu.VMEM((_TBL,), jnp.int32),
                pltpu.VMEM((2 * _PER_W,), jnp.int32),
            ],
            compiler_params=pltpu.CompilerParams(needs_layout_passes=False),
        )
        return run(flat_pairs, vocab_pairs)


def kernel(inputs, vocabulary):
    vocab_padded = jnp.concatenate(
        [vocabulary, jnp.full((_VOCAB_PAD - _VOCAB_SIZE,), _TBL - 1, vocabulary.dtype)]
    )
    flat_pairs = lax.bitcast_convert_type(inputs, jnp.int32).reshape(-1)
    vocab_pairs = lax.bitcast_convert_type(vocab_padded, jnp.int32).reshape(-1)
    out_pairs = _lookup(flat_pairs, vocab_pairs)
    return lax.bitcast_convert_type(
        out_pairs.reshape(*inputs.shape, 2), jnp.int64
    )


# speedup vs baseline: 9.5465x; 9.5465x over previous
import jax
import jax.numpy as jnp
from jax import lax
from jax.experimental import pallas as pl
from jax.experimental.pallas import tpu as pltpu
from jax.experimental.pallas import tpu_sc as plsc

_VOCAB_SIZE = 1000
_TBL = 2048
_N = 4096 * 20
_NW = 32
_PER_W = _N // _NW
_L = 16
_FULL = _VOCAB_SIZE // _L


def _lookup_body(inp_hbm, vocab_hbm, out_hbm, inp_v, vocab_v, inv_v, out_v,
                 inp_sem, vocab_sem):
    wid = lax.axis_index("s") * 2 + lax.axis_index("c")
    base = wid * _PER_W
    inp_dma = pltpu.async_copy(inp_hbm.at[pl.ds(base, _PER_W)], inp_v, inp_sem)
    vocab_dma = pltpu.async_copy(vocab_hbm, vocab_v, vocab_sem)

    lane = lax.iota(jnp.int32, _L)
    oov = jnp.full((_L,), _VOCAB_SIZE, jnp.int32)

    def init_step(i, carry):
        inv_v[pl.ds(i * _L, _L)] = oov
        return carry

    lax.fori_loop(0, _TBL // _L, init_step, 0, unroll=8)
    vocab_dma.wait()

    def scatter_step(j, carry):
        keys = vocab_v[pl.ds(j * _L, _L)]
        plsc.store_scatter(inv_v, [keys], lane + j * _L)
        return carry

    lax.fori_loop(0, _FULL, scatter_step, 0, unroll=8)
    tail_mask = lane < (_VOCAB_SIZE - _FULL * _L)
    tail_idx = jnp.minimum(lane + _FULL * _L, _VOCAB_SIZE - 1)
    tail_keys = plsc.load_gather(vocab_v, [tail_idx])
    plsc.store_scatter(inv_v, [tail_keys], lane + _FULL * _L, mask=tail_mask)

    inp_dma.wait()

    def gather_step(i, carry):
        off = i * _L
        toks = inp_v[pl.ds(off, _L)]
        out_v[pl.ds(off, _L)] = plsc.load_gather(inv_v, [toks])
        return carry

    lax.fori_loop(0, _PER_W // _L, gather_step, 0, unroll=8)

    pltpu.sync_copy(out_v, out_hbm.at[pl.ds(base, _PER_W)])


@jax.jit
def _lookup(flat_inputs, vocab):
    with jax.enable_x64(False):
        mesh = plsc.VectorSubcoreMesh(core_axis_name="c", subcore_axis_name="s")
        run = pl.kernel(
            _lookup_body,
            out_type=jax.ShapeDtypeStruct((_N,), jnp.int32),
            mesh=mesh,
            scratch_types=[
                pltpu.VMEM((_PER_W,), jnp.int32),
                pltpu.VMEM((_VOCAB_SIZE,), jnp.int32),
                pltpu.VMEM((_TBL,), jnp.int32),
                pltpu.VMEM((_PER_W,), jnp.int32),
                pltpu.SemaphoreType.DMA,
                pltpu.SemaphoreType.DMA,
            ],
            compiler_params=pltpu.CompilerParams(needs_layout_passes=False),
        )
        return run(flat_inputs, vocab)


def kernel(inputs, vocabulary):
    flat = inputs.astype(jnp.int32).T.reshape(-1)
    vocab = vocabulary.astype(jnp.int32)
    out = _lookup(flat, vocab)
    return out.reshape(inputs.shape[::-1]).T.astype(jnp.int64)


# speedup vs baseline: 9.5918x; 1.0047x over previous
import jax
import jax.numpy as jnp
from jax import lax
from jax.experimental import pallas as pl
from jax.experimental.pallas import tpu as pltpu
from jax.experimental.pallas import tpu_sc as plsc

_VOCAB_SIZE = 1000
_TBL = 2048
_N = 4096 * 20
_NW = 32
_PER_W = _N // _NW
_L = 16
_FULL = _VOCAB_SIZE // _L


def _lookup_body(inp_hbm, vocab_hbm, out_hbm, inp_v, vocab_v, inv_v, out_v,
                 inp_sem, vocab_sem):
    wid = lax.axis_index("s") * 2 + lax.axis_index("c")
    base = wid * _PER_W
    inp_dma = pltpu.async_copy(inp_hbm.at[pl.ds(base, _PER_W)], inp_v, inp_sem)
    vocab_dma = pltpu.async_copy(vocab_hbm, vocab_v, vocab_sem)

    lane = lax.iota(jnp.int32, _L)
    oov = jnp.full((_L,), _VOCAB_SIZE, jnp.int32)

    def init_step(i, carry):
        inv_v[pl.ds(i * _L, _L)] = oov
        return carry

    lax.fori_loop(0, _TBL // _L, init_step, 0, unroll=8)
    vocab_dma.wait()

    def scatter_step(j, carry):
        keys = plsc.bitcast(vocab_v[pl.ds(j * _L, _L)], jnp.int32)
        plsc.store_scatter(inv_v, [keys], lane + j * _L)
        return carry

    lax.fori_loop(0, _FULL, scatter_step, 0, unroll=8)
    tail_keys = plsc.bitcast(vocab_v[pl.ds(_VOCAB_SIZE - _L, _L)], jnp.int32)
    plsc.store_scatter(inv_v, [tail_keys], lane + (_VOCAB_SIZE - _L))

    inp_dma.wait()

    def gather_step(i, carry):
        off = i * _L
        toks = plsc.bitcast(inp_v[pl.ds(off, _L)], jnp.int32)
        out_v[pl.ds(off, _L)] = plsc.bitcast(
            plsc.load_gather(inv_v, [toks]), jnp.uint32)
        return carry

    lax.fori_loop(0, _PER_W // _L, gather_step, 0, unroll=8)

    pltpu.sync_copy(out_v, out_hbm.at[pl.ds(base, _PER_W)])


@jax.jit
def _lookup(flat_inputs, vocab):
    with jax.enable_x64(False):
        mesh = plsc.VectorSubcoreMesh(core_axis_name="c", subcore_axis_name="s")
        run = pl.kernel(
            _lookup_body,
            out_type=jax.ShapeDtypeStruct((_N,), jnp.uint32),
            mesh=mesh,
            scratch_types=[
                pltpu.VMEM((_PER_W,), jnp.uint32),
                pltpu.VMEM((_VOCAB_SIZE,), jnp.uint32),
                pltpu.VMEM((_TBL,), jnp.int32),
                pltpu.VMEM((_PER_W,), jnp.uint32),
                pltpu.SemaphoreType.DMA,
                pltpu.SemaphoreType.DMA,
            ],
            compiler_params=pltpu.CompilerParams(needs_layout_passes=False),
        )
        return run(flat_inputs, vocab)


def kernel(inputs, vocabulary):
    flat = inputs.astype(jnp.uint32).T.reshape(-1)
    vocab = vocabulary.astype(jnp.uint32)
    out = _lookup(flat, vocab)
    return out.reshape(inputs.shape[::-1]).T.astype(jnp.int64)
